# Initial kernel scaffold; baseline (speedup 1.0000x reference)
import jax
import jax.numpy as jnp
from jax.experimental import pallas as pl


def kernel(x, edge_index, batch, num_subgraphs, subgraph_batch, W1, b1, W2, b2, gamma, beta, eps_gin):
    raise NotImplementedError("write your pallas kernel here")


# speedup vs baseline: 5.3633x; 5.3633x over previous
import functools

import jax
import jax.numpy as jnp
from jax import lax
from jax.experimental import pallas as pl
from jax.experimental.pallas import tpu as pltpu
from jax.experimental.pallas import tpu_sc as plsc

NC = 2
NS = 16
NW = NC * NS
CHUNK = 128


def _round_up(a, b):
  return (a + b - 1) // b * b


def _seg_pad(n_seg):
  return _round_up(n_seg + 1, NS * 8)


def _make_scatter_add(n_seg, width, epad):
  ept = epad // NW
  cpt = ept // CHUNK
  n_seg_pad = _seg_pad(n_seg)
  rpt = n_seg_pad // NS

  mesh = plsc.VectorSubcoreMesh(core_axis_name="c", subcore_axis_name="s")

  @functools.partial(
      pl.kernel,
      mesh=mesh,
      out_type=jax.ShapeDtypeStruct((NC, n_seg_pad, width), jnp.float32),
      scratch_types=[
          pltpu.VMEM((CHUNK,), jnp.int32),
          pltpu.VMEM((CHUNK,), jnp.int32),
          pltpu.VMEM((CHUNK, width), jnp.float32),
          pltpu.VMEM_SHARED((n_seg_pad, width), jnp.float32),
          pltpu.SemaphoreType.DMA,
      ],
  )
  def sc_kernel(src_h, dst_h, h_h, out_h, src_v, dst_v, rows_v, acc_sh, sem):
    cid = lax.axis_index("c")
    sid = lax.axis_index("s")
    wid = cid * NS + sid

    def zrow(i, _):
      def zcol(j, _):
        rows_v[i, pl.ds(j * 16, 16)] = jnp.zeros((16,), jnp.float32)
        return 0
      return lax.fori_loop(0, width // 16, zcol, 0)
    lax.fori_loop(0, CHUNK, zrow, 0)

    r0 = sid * rpt
    off = 0
    while off < rpt:
      c = min(CHUNK, rpt - off)
      pltpu.sync_copy(rows_v.at[pl.ds(0, c)], acc_sh.at[pl.ds(r0 + off, c)])
      off += c
    plsc.subcore_barrier()

    ebase = wid * ept

    def step(i, _):
      base = pl.multiple_of(ebase + i * CHUNK, 8)
      pltpu.sync_copy(src_h.at[pl.ds(base, CHUNK)], src_v)
      pltpu.sync_copy(dst_h.at[pl.ds(base, CHUNK)], dst_v)
      pltpu.async_copy(h_h.at[src_v], rows_v, sem).wait()
      pltpu.sync_copy(rows_v, acc_sh.at[dst_v], add=True)
      return 0
    lax.fori_loop(0, cpt, step, 0)

    plsc.subcore_barrier()
    pltpu.sync_copy(acc_sh.at[pl.ds(r0, rpt)], out_h.at[cid, pl.ds(r0, rpt)])

  return sc_kernel


def _tc_layer(h, part, W1l, b1l, W2l, b2l, gammal, betal, epsl, last, n, d):
  out_rows = n + 128 if last else n

  def body(h_ref, p_ref, w1_ref, b1_ref, w2_ref, b2_ref, g_ref, be_ref,
           e_ref, out_ref):
    eps = e_ref[0, 0]
    z = (1.0 + eps) * h_ref[...] + p_ref[0] + p_ref[1]
    y = jnp.dot(z, w1_ref[...], preferred_element_type=jnp.float32)
    y = jnp.maximum(y + b1_ref[...], 0.0)
    z2 = jnp.dot(y, w2_ref[...], preferred_element_type=jnp.float32)
    z2 = z2 + b2_ref[...]
    mu = jnp.mean(z2, axis=0, keepdims=True)
    dlt = z2 - mu
    var = jnp.mean(dlt * dlt, axis=0, keepdims=True)
    zn = dlt * lax.rsqrt(var + 1e-5) * g_ref[...] + be_ref[...]
    if last:
      out_ref[...] = jnp.concatenate(
          [zn, jnp.ones((128, d), jnp.float32)], axis=0)
    else:
      out_ref[...] = jnp.maximum(zn, 0.0)

  return pl.pallas_call(
      body,
      out_shape=jax.ShapeDtypeStruct((out_rows, d), jnp.float32),
  )(h, part, W1l, b1l.reshape(1, -1), W2l, b2l.reshape(1, -1),
    gammal.reshape(1, -1), betal.reshape(1, -1),
    epsl.reshape(1, 1))


def _tc_finalize(parts, s, d):
  def body(p_ref, out_ref):
    sums = p_ref[0, :s, :] + p_ref[1, :s, :]
    counts = p_ref[0, s:, :] + p_ref[1, s:, :]
    out_ref[...] = sums / jnp.clip(counts, 1.0, None)

  return pl.pallas_call(
      body,
      out_shape=jax.ShapeDtypeStruct((s, d), jnp.float32),
  )(parts)


def kernel(x, edge_index, batch, num_subgraphs, subgraph_batch,
           W1, b1, W2, b2, gamma, beta, eps_gin):
  n, d = x.shape
  e = edge_index.shape[1]
  num_layers = W1.shape[0]
  s = 2048

  epad = _round_up(e, NW * CHUNK)
  npd = _seg_pad(n)
  pad_i = jnp.arange(epad - e, dtype=jnp.int32)
  src = jnp.concatenate([edge_index[0], pad_i % n])
  dst = jnp.concatenate([edge_index[1], n + pad_i % (npd - n)])
  agg_call = _make_scatter_add(n, d, epad)

  h = x
  for l in range(num_layers):
    part = agg_call(src, dst, h)[:, :n, :]
    h = _tc_layer(h, part, W1[l], b1[l], W2[l], b2[l], gamma[l], beta[l],
                  eps_gin[l], l == num_layers - 1, n, d)

  sg = subgraph_batch.astype(jnp.int32)
  ppad = _round_up(2 * n, NW * CHUNK)
  spd = _seg_pad(2 * s)
  ar = jnp.arange(n, dtype=jnp.int32)
  pad_j = jnp.arange(ppad - 2 * n, dtype=jnp.int32)
  psrc = jnp.concatenate([ar, n + ar % 128, pad_j % n])
  pdst = jnp.concatenate([sg, s + sg, 2 * s + pad_j % (spd - 2 * s)])
  pool_call = _make_scatter_add(2 * s, d, ppad)
  parts = pool_call(psrc, pdst, h)[:, :2 * s, :]
  return _tc_finalize(parts, s, d)
